# baseline (speedup 1.0000x reference)
import functools

import jax
import jax.numpy as jnp
from jax import lax
from jax.experimental import pallas as pl
from jax.experimental.pallas import tpu as pltpu
from jax.experimental.pallas import tpu_sc as plsc

B = 16384
NE = 1000000
NR = 1000
D = 64
EPS = 1e-12

NC = 2
NS = 16
NW = NC * NS
ROWS_PER_W = B // NW
C = 128
NCHUNK = ROWS_PER_W // C

_GATHER_DN = lax.GatherDimensionNumbers(
    offset_dims=(), collapsed_slice_dims=(0,), start_index_map=(0,))


def _shuffle(v, perm):
    return lax.gather(v, perm, _GATHER_DN, (1,),
                      mode=lax.GatherScatterMode.PROMISE_IN_BOUNDS)


def _hsum16(v, perms):
    for p in perms:
        v = v + _shuffle(v, p)
    return v


def _rsqrt(n):
    i = plsc.bitcast(n, jnp.int32)
    i = 0x5F3759DF - lax.shift_right_logical(i, 1)
    y = plsc.bitcast(i, jnp.float32)
    h = 0.5 * n
    for _ in range(3):
        y = y * (1.5 - h * y * y)
    return y


def _make_kernel():
    mesh = plsc.VectorSubcoreMesh(core_axis_name="c", subcore_axis_name="s")

    @functools.partial(
        pl.kernel,
        mesh=mesh,
        compiler_params=pltpu.CompilerParams(
            needs_layout_passes=False, use_tc_tiling_on_sc=True),
        out_type=jax.ShapeDtypeStruct((D, B), jnp.float32),
        scratch_types=[
            pltpu.VMEM((C,), jnp.int32),
            pltpu.VMEM((C,), jnp.int32),
            pltpu.VMEM((C,), jnp.int32),
            pltpu.VMEM((C,), jnp.int32),
            pltpu.VMEM((C,), jnp.int32),
            pltpu.VMEM((C,), jnp.int32),
            pltpu.VMEM((C, 128), jnp.float32),
            pltpu.VMEM((C, 128), jnp.float32),
            pltpu.VMEM((C, 128), jnp.float32),
            pltpu.VMEM((D, ROWS_PER_W), jnp.float32),
            pltpu.SemaphoreType.DMA,
        ],
    )
    def emodel(s_hbm, o_hbm, r_hbm, e2_hbm, r2_hbm, out_hbm,
               sidx, oidx, ridx, sh, oh, rh, se_v, oe_v, re_v, out_v, sem):
        wid = lax.axis_index("s") * NC + lax.axis_index("c")

        lane = lax.iota(jnp.int32, 16)
        perms = [((lane + k) & 15).reshape(16, 1) for k in (8, 4, 2, 1)]
        cvecs = [16 * q + lane for q in range(4)]
        zeros16 = jnp.zeros((16,), jnp.int32)

        def do_chunk(ch, carry):
            base = wid * ROWS_PER_W + ch * C
            pltpu.sync_copy(s_hbm.at[pl.ds(base, C)], sidx)
            pltpu.sync_copy(o_hbm.at[pl.ds(base, C)], oidx)
            pltpu.sync_copy(r_hbm.at[pl.ds(base, C)], ridx)
            for g in range(C // 16):
                for idxb, hb in ((sidx, sh), (oidx, oh), (ridx, rh)):
                    v = idxb[pl.ds(16 * g, 16)]
                    hb[pl.ds(16 * g, 16)] = (v & 1) * 64
                    idxb[pl.ds(16 * g, 16)] = lax.shift_right_logical(v, 1)
            cp_s = pltpu.async_copy(e2_hbm.at[sidx], se_v, sem)
            cp_o = pltpu.async_copy(e2_hbm.at[oidx], oe_v, sem)
            cp_r = pltpu.async_copy(r2_hbm.at[ridx], re_v, sem)
            cp_s.wait()
            cp_o.wait()
            cp_r.wait()

            def row_group(g, carry2):
                hsv = sh[pl.ds(16 * g, 16)]
                hov = oh[pl.ds(16 * g, 16)]
                hrv = rh[pl.ds(16 * g, 16)]
                for j in range(16):
                    rr = 16 * g + j
                    hs, ho, hr = hsv[j], hov[j], hrv[j]
                    sv = [se_v[rr, pl.ds(hs + 16 * k, 16)] for k in range(4)]
                    ov = [oe_v[rr, pl.ds(ho + 16 * k, 16)] for k in range(4)]
                    rv = [re_v[rr, pl.ds(hr + 16 * k, 16)] for k in range(4)]

                    ss = sv[0] * sv[0] + sv[1] * sv[1] + sv[2] * sv[2] + sv[3] * sv[3]
                    so = ov[0] * ov[0] + ov[1] * ov[1] + ov[2] * ov[2] + ov[3] * ov[3]
                    sr = rv[0] * rv[0] + rv[1] * rv[1] + rv[2] * rv[2] + rv[3] * rv[3]
                    inv_s = _rsqrt(jnp.maximum(_hsum16(ss, perms), 1e-24))
                    inv_o = _rsqrt(jnp.maximum(_hsum16(so, perms), 1e-24))
                    inv_r = _rsqrt(jnp.maximum(_hsum16(sr, perms), 1e-24))

                    cv = [sv[k] * inv_s + rv[k] * inv_r - ov[k] * inv_o
                          for k in range(4)]
                    a = (jnp.abs(cv[0]) + jnp.abs(cv[1])
                         + jnp.abs(cv[2]) + jnp.abs(cv[3]))
                    inv_l1 = 1.0 / jnp.maximum(_hsum16(a, perms), EPS)
                    ocol = zeros16 + (ch * C + rr)
                    for q in range(4):
                        plsc.store_scatter(out_v, [cvecs[q], ocol],
                                           cv[q] * inv_l1)
                return carry2

            lax.fori_loop(0, C // 16, row_group, 0)
            return carry

        lax.fori_loop(0, NCHUNK, do_chunk, 0)
        pltpu.sync_copy(out_v, out_hbm.at[:, pl.ds(wid * ROWS_PER_W,
                                                   ROWS_PER_W)])

    return emodel


_emodel = _make_kernel()


def kernel(s, r, o, e_emb, r_emb):
    s = jnp.asarray(s, jnp.int32)
    o = jnp.asarray(o, jnp.int32)
    r = jnp.asarray(r, jnp.int32)
    e2 = e_emb.reshape(NE // 2, 2 * D)
    r2 = r_emb.reshape(NR // 2, 2 * D)
    out_t = _emodel(s, o, r, e2, r2)
    return out_t.T


# speedup vs baseline: 1.6780x; 1.6780x over previous
import functools

import jax
import jax.numpy as jnp
from jax import lax
from jax.experimental import pallas as pl
from jax.experimental.pallas import tpu as pltpu
from jax.experimental.pallas import tpu_sc as plsc

B = 16384
NE = 1000000
NR = 1000
D = 64
EPS = 1e-12

NC = 2
NS = 16
NW = NC * NS
ROWS_PER_W = B // NW
C = 128
NCHUNK = ROWS_PER_W // C


_GATHER_DN = lax.GatherDimensionNumbers(
    offset_dims=(), collapsed_slice_dims=(0,), start_index_map=(0,))


def _shuffle(v, perm):
    return lax.gather(v, perm, _GATHER_DN, (1,),
                      mode=lax.GatherScatterMode.PROMISE_IN_BOUNDS)


def _hsum16(v, perms):
    for p in perms:
        v = v + _shuffle(v, p)
    return v


def _rsqrt(n):
    i = plsc.bitcast(n, jnp.int32)
    i = 0x5F3759DF - lax.shift_right_logical(i, 1)
    y = plsc.bitcast(i, jnp.float32)
    h = 0.5 * n
    for _ in range(3):
        y = y * (1.5 - h * y * y)
    return y


def _make_kernel():
    mesh = plsc.VectorSubcoreMesh(core_axis_name="c", subcore_axis_name="s")

    @functools.partial(
        pl.kernel,
        mesh=mesh,
        compiler_params=pltpu.CompilerParams(
            needs_layout_passes=False, use_tc_tiling_on_sc=True),
        out_type=jax.ShapeDtypeStruct((D, B), jnp.float32),
        scratch_types=[
            pltpu.VMEM((C,), jnp.int32),
            pltpu.VMEM((C,), jnp.int32),
            pltpu.VMEM((C,), jnp.int32),
            pltpu.VMEM((C, D), jnp.float32),
            pltpu.VMEM((C, D), jnp.float32),
            pltpu.VMEM((C, D), jnp.float32),
            pltpu.VMEM((D, ROWS_PER_W), jnp.float32),
            pltpu.SemaphoreType.DMA,
        ],
    )
    def emodel(s_hbm, o_hbm, r_hbm, e_hbm, rel_hbm, out_hbm,
               sidx, oidx, ridx, se_v, oe_v, re_v, out_v, sem):
        wid = lax.axis_index("s") * NC + lax.axis_index("c")

        lane = lax.iota(jnp.int32, 16)
        perms = [((lane + k) & 15).reshape(16, 1) for k in (8, 4, 2, 1)]
        cvecs = [16 * q + lane for q in range(4)]
        zeros16 = jnp.zeros((16,), jnp.int32)

        def do_chunk(ch, _):
            base = wid * ROWS_PER_W + ch * C
            pltpu.sync_copy(s_hbm.at[pl.ds(base, C)], sidx)
            pltpu.sync_copy(o_hbm.at[pl.ds(base, C)], oidx)
            pltpu.sync_copy(r_hbm.at[pl.ds(base, C)], ridx)

            def issue(g, carry):
                svec = sidx[pl.ds(16 * g, 16)]
                ovec = oidx[pl.ds(16 * g, 16)]
                rvec = ridx[pl.ds(16 * g, 16)]
                for j in range(16):
                    i = 16 * g + j
                    pltpu.async_copy(e_hbm.at[svec[j]], se_v.at[i], sem)
                    pltpu.async_copy(e_hbm.at[ovec[j]], oe_v.at[i], sem)
                    pltpu.async_copy(rel_hbm.at[rvec[j]], re_v.at[i], sem)
                return carry

            lax.fori_loop(0, C // 16, issue, 0)
            pltpu.make_async_copy(e_hbm.at[pl.ds(0, C)], se_v, sem).wait()
            pltpu.make_async_copy(e_hbm.at[pl.ds(0, C)], oe_v, sem).wait()
            pltpu.make_async_copy(e_hbm.at[pl.ds(0, C)], re_v, sem).wait()

            def row(rr, carry):
                sv = [se_v[rr, pl.ds(16 * k, 16)] for k in range(4)]
                ov = [oe_v[rr, pl.ds(16 * k, 16)] for k in range(4)]
                rv = [re_v[rr, pl.ds(16 * k, 16)] for k in range(4)]

                ss = sv[0] * sv[0] + sv[1] * sv[1] + sv[2] * sv[2] + sv[3] * sv[3]
                so = ov[0] * ov[0] + ov[1] * ov[1] + ov[2] * ov[2] + ov[3] * ov[3]
                sr = rv[0] * rv[0] + rv[1] * rv[1] + rv[2] * rv[2] + rv[3] * rv[3]
                inv_s = _rsqrt(jnp.maximum(_hsum16(ss, perms), 1e-24))
                inv_o = _rsqrt(jnp.maximum(_hsum16(so, perms), 1e-24))
                inv_r = _rsqrt(jnp.maximum(_hsum16(sr, perms), 1e-24))

                c = [sv[k] * inv_s + rv[k] * inv_r - ov[k] * inv_o
                     for k in range(4)]
                a = jnp.abs(c[0]) + jnp.abs(c[1]) + jnp.abs(c[2]) + jnp.abs(c[3])
                inv_l1 = 1.0 / jnp.maximum(_hsum16(a, perms), EPS)
                ocol = zeros16 + (ch * C + rr)
                for k in range(4):
                    plsc.store_scatter(out_v, [cvecs[k], ocol],
                                       c[k] * inv_l1)
                return carry

            lax.fori_loop(0, C, row, 0)
            return _

        lax.fori_loop(0, NCHUNK, do_chunk, 0)
        pltpu.sync_copy(out_v, out_hbm.at[:, pl.ds(wid * ROWS_PER_W,
                                                   ROWS_PER_W)])

    return emodel


_emodel = _make_kernel()


def kernel(s, r, o, e_emb, r_emb):
    s = jnp.asarray(s, jnp.int32)
    o = jnp.asarray(o, jnp.int32)
    r = jnp.asarray(r, jnp.int32)
    return _emodel(s, o, r, e_emb, r_emb).T


# speedup vs baseline: 1.7530x; 1.0447x over previous
import functools

import jax
import jax.numpy as jnp
from jax import lax
from jax.experimental import pallas as pl
from jax.experimental.pallas import tpu as pltpu
from jax.experimental.pallas import tpu_sc as plsc

B = 16384
NE = 1000000
NR = 1000
D = 64
EPS = 1e-12

NC = 2
NS = 16
NW = NC * NS
ROWS_PER_W = B // NW
C = 128
NCHUNK = ROWS_PER_W // C


_GATHER_DN = lax.GatherDimensionNumbers(
    offset_dims=(), collapsed_slice_dims=(0,), start_index_map=(0,))


def _shuffle(v, perm):
    return lax.gather(v, perm, _GATHER_DN, (1,),
                      mode=lax.GatherScatterMode.PROMISE_IN_BOUNDS)


def _hsum16(v, perms):
    for p in perms:
        v = v + _shuffle(v, p)
    return v


def _rsqrt(n):
    i = plsc.bitcast(n, jnp.int32)
    i = 0x5F3759DF - lax.shift_right_logical(i, 1)
    y = plsc.bitcast(i, jnp.float32)
    h = 0.5 * n
    for _ in range(3):
        y = y * (1.5 - h * y * y)
    return y


def _make_kernel():
    mesh = plsc.VectorSubcoreMesh(core_axis_name="c", subcore_axis_name="s")

    @functools.partial(
        pl.kernel,
        mesh=mesh,
        compiler_params=pltpu.CompilerParams(
            needs_layout_passes=False, use_tc_tiling_on_sc=True),
        out_type=jax.ShapeDtypeStruct((B, D), jnp.float32),
        scratch_types=[
            pltpu.VMEM((C,), jnp.int32),
            pltpu.VMEM((C,), jnp.int32),
            pltpu.VMEM((C,), jnp.int32),
            pltpu.VMEM((C, D), jnp.float32),
            pltpu.VMEM((C, D), jnp.float32),
            pltpu.VMEM((C, D), jnp.float32),
            pltpu.VMEM((C, D), jnp.float32),
            pltpu.SemaphoreType.DMA,
        ],
    )
    def emodel(s_hbm, o_hbm, r_hbm, e_hbm, rel_hbm, out_hbm,
               sidx, oidx, ridx, se_v, oe_v, re_v, out_v, sem):
        wid = lax.axis_index("s") * NC + lax.axis_index("c")

        lane = lax.iota(jnp.int32, 16)
        perms = [((lane + k) & 15).reshape(16, 1) for k in (8, 4, 2, 1)]

        def do_chunk(ch, _):
            base = wid * ROWS_PER_W + ch * C
            pltpu.sync_copy(s_hbm.at[pl.ds(base, C)], sidx)
            pltpu.sync_copy(o_hbm.at[pl.ds(base, C)], oidx)
            pltpu.sync_copy(r_hbm.at[pl.ds(base, C)], ridx)

            def issue(g, carry):
                svec = sidx[pl.ds(16 * g, 16)]
                ovec = oidx[pl.ds(16 * g, 16)]
                rvec = ridx[pl.ds(16 * g, 16)]
                for j in range(16):
                    i = 16 * g + j
                    pltpu.async_copy(e_hbm.at[svec[j]], se_v.at[i], sem)
                    pltpu.async_copy(e_hbm.at[ovec[j]], oe_v.at[i], sem)
                    pltpu.async_copy(rel_hbm.at[rvec[j]], re_v.at[i], sem)
                return carry

            lax.fori_loop(0, C // 16, issue, 0)
            pltpu.make_async_copy(e_hbm.at[pl.ds(0, C)], se_v, sem).wait()
            pltpu.make_async_copy(e_hbm.at[pl.ds(0, C)], oe_v, sem).wait()
            pltpu.make_async_copy(e_hbm.at[pl.ds(0, C)], re_v, sem).wait()

            def row(rr, carry):
                sv = [se_v[rr, pl.ds(16 * k, 16)] for k in range(4)]
                ov = [oe_v[rr, pl.ds(16 * k, 16)] for k in range(4)]
                rv = [re_v[rr, pl.ds(16 * k, 16)] for k in range(4)]

                ss = sv[0] * sv[0] + sv[1] * sv[1] + sv[2] * sv[2] + sv[3] * sv[3]
                so = ov[0] * ov[0] + ov[1] * ov[1] + ov[2] * ov[2] + ov[3] * ov[3]
                sr = rv[0] * rv[0] + rv[1] * rv[1] + rv[2] * rv[2] + rv[3] * rv[3]
                inv_s = _rsqrt(jnp.maximum(_hsum16(ss, perms), 1e-24))
                inv_o = _rsqrt(jnp.maximum(_hsum16(so, perms), 1e-24))
                inv_r = _rsqrt(jnp.maximum(_hsum16(sr, perms), 1e-24))

                c = [sv[k] * inv_s + rv[k] * inv_r - ov[k] * inv_o
                     for k in range(4)]
                a = jnp.abs(c[0]) + jnp.abs(c[1]) + jnp.abs(c[2]) + jnp.abs(c[3])
                inv_l1 = 1.0 / jnp.maximum(_hsum16(a, perms), EPS)
                for k in range(4):
                    out_v[rr, pl.ds(16 * k, 16)] = c[k] * inv_l1
                return carry

            lax.fori_loop(0, C, row, 0)
            pltpu.sync_copy(out_v, out_hbm.at[pl.ds(base, C)])
            return _

        lax.fori_loop(0, NCHUNK, do_chunk, 0)

    return emodel


_emodel = _make_kernel()


def kernel(s, r, o, e_emb, r_emb):
    s = jnp.asarray(s, jnp.int32)
    o = jnp.asarray(o, jnp.int32)
    r = jnp.asarray(r, jnp.int32)
    return _emodel(s, o, r, e_emb, r_emb)


# speedup vs baseline: 2.5142x; 1.4342x over previous
import functools

import jax
import jax.numpy as jnp
from jax import lax
from jax.experimental import pallas as pl
from jax.experimental.pallas import tpu as pltpu
from jax.experimental.pallas import tpu_sc as plsc

B = 16384
NE = 1000000
NR = 1000
D = 64
EPS = 1e-12

NC = 2
NS = 16
NW = NC * NS
ROWS_PER_W = B // NW
C = 128
NCHUNK = ROWS_PER_W // C


_GATHER_DN = lax.GatherDimensionNumbers(
    offset_dims=(), collapsed_slice_dims=(0,), start_index_map=(0,))


def _shuffle(v, perm):
    return lax.gather(v, perm, _GATHER_DN, (1,),
                      mode=lax.GatherScatterMode.PROMISE_IN_BOUNDS)


def _hsum16(v, perms):
    for p in perms:
        v = v + _shuffle(v, p)
    return v


def _rsqrt(n):
    i = plsc.bitcast(n, jnp.int32)
    i = 0x5F3759DF - lax.shift_right_logical(i, 1)
    y = plsc.bitcast(i, jnp.float32)
    h = 0.5 * n
    for _ in range(3):
        y = y * (1.5 - h * y * y)
    return y


def _make_kernel():
    mesh = plsc.VectorSubcoreMesh(core_axis_name="c", subcore_axis_name="s")

    @functools.partial(
        pl.kernel,
        mesh=mesh,
        compiler_params=pltpu.CompilerParams(
            needs_layout_passes=False, use_tc_tiling_on_sc=True),
        out_type=jax.ShapeDtypeStruct((B, D), jnp.float32),
        scratch_types=[
            pltpu.VMEM((C,), jnp.int32),
            pltpu.VMEM((C,), jnp.int32),
            pltpu.VMEM((C,), jnp.int32),
            pltpu.VMEM((C, D), jnp.float32),
            pltpu.VMEM((C, D), jnp.float32),
            pltpu.VMEM((C, D), jnp.float32),
            pltpu.VMEM((C, D), jnp.float32),
            pltpu.SemaphoreType.DMA,
        ],
    )
    def emodel(s_hbm, o_hbm, r_hbm, e_hbm, rel_hbm, out_hbm,
               sidx, oidx, ridx, se_v, oe_v, re_v, out_v, sem):
        wid = lax.axis_index("s") * NC + lax.axis_index("c")

        lane = lax.iota(jnp.int32, 16)
        perms = [((lane + k) & 15).reshape(16, 1) for k in (8, 4, 2, 1)]

        def do_chunk(ch, _):
            base = wid * ROWS_PER_W + ch * C
            pltpu.sync_copy(s_hbm.at[pl.ds(base, C)], sidx)
            pltpu.sync_copy(o_hbm.at[pl.ds(base, C)], oidx)
            pltpu.sync_copy(r_hbm.at[pl.ds(base, C)], ridx)

            def issue(g, carry):
                svec = sidx[pl.ds(16 * g, 16)]
                ovec = oidx[pl.ds(16 * g, 16)]
                rvec = ridx[pl.ds(16 * g, 16)]
                for j in range(16):
                    i = 16 * g + j
                    sj, oj, rj = svec[j], ovec[j], rvec[j]
                    pltpu.async_copy(
                        e_hbm.at[lax.shift_right_logical(sj, 1), sj & 1],
                        se_v.at[i], sem)
                    pltpu.async_copy(
                        e_hbm.at[lax.shift_right_logical(oj, 1), oj & 1],
                        oe_v.at[i], sem)
                    pltpu.async_copy(
                        rel_hbm.at[lax.shift_right_logical(rj, 1), rj & 1],
                        re_v.at[i], sem)
                return carry

            lax.fori_loop(0, C // 16, issue, 0)
            pltpu.make_async_copy(e_hbm.at[pl.ds(0, C // 2)], se_v.reshape(C // 2, 2, D), sem).wait()
            pltpu.make_async_copy(e_hbm.at[pl.ds(0, C // 2)], oe_v.reshape(C // 2, 2, D), sem).wait()
            pltpu.make_async_copy(e_hbm.at[pl.ds(0, C // 2)], re_v.reshape(C // 2, 2, D), sem).wait()

            def row(rr, carry):
                sv = [se_v[rr, pl.ds(16 * k, 16)] for k in range(4)]
                ov = [oe_v[rr, pl.ds(16 * k, 16)] for k in range(4)]
                rv = [re_v[rr, pl.ds(16 * k, 16)] for k in range(4)]

                ss = sv[0] * sv[0] + sv[1] * sv[1] + sv[2] * sv[2] + sv[3] * sv[3]
                so = ov[0] * ov[0] + ov[1] * ov[1] + ov[2] * ov[2] + ov[3] * ov[3]
                sr = rv[0] * rv[0] + rv[1] * rv[1] + rv[2] * rv[2] + rv[3] * rv[3]
                inv_s = _rsqrt(jnp.maximum(_hsum16(ss, perms), 1e-24))
                inv_o = _rsqrt(jnp.maximum(_hsum16(so, perms), 1e-24))
                inv_r = _rsqrt(jnp.maximum(_hsum16(sr, perms), 1e-24))

                c = [sv[k] * inv_s + rv[k] * inv_r - ov[k] * inv_o
                     for k in range(4)]
                a = jnp.abs(c[0]) + jnp.abs(c[1]) + jnp.abs(c[2]) + jnp.abs(c[3])
                inv_l1 = 1.0 / jnp.maximum(_hsum16(a, perms), EPS)
                for k in range(4):
                    out_v[rr, pl.ds(16 * k, 16)] = c[k] * inv_l1
                return carry

            lax.fori_loop(0, C, row, 0)
            pltpu.sync_copy(out_v, out_hbm.at[pl.ds(base, C)])
            return _

        lax.fori_loop(0, NCHUNK, do_chunk, 0)

    return emodel


_emodel = _make_kernel()


def kernel(s, r, o, e_emb, r_emb):
    s = jnp.asarray(s, jnp.int32)
    o = jnp.asarray(o, jnp.int32)
    r = jnp.asarray(r, jnp.int32)
    e3 = e_emb.reshape(NE // 2, 2, D)
    r3 = r_emb.reshape(NR // 2, 2, D)
    return _emodel(s, o, r, e3, r3)
